# Initial kernel scaffold; baseline (speedup 1.0000x reference)
import jax
import jax.numpy as jnp
from jax.experimental import pallas as pl


def kernel(x, enc_w1, enc_b1, enc_w2, enc_b2, enc_w3, enc_b3, fc_w, fc_b, dec_w1v, dec_w1f, dec_b1, dec_w2, dec_b2, dec_w3, dec_b3, dec_w4, dec_b4, vertex):
    raise NotImplementedError("write your pallas kernel here")


# speedup vs baseline: 1.1702x; 1.1702x over previous
import jax
import jax.numpy as jnp
from jax.experimental import pallas as pl
from jax.experimental.pallas import tpu as pltpu

F32 = jnp.float32


def _dot_tb(a, b):
    return jax.lax.dot_general(a, b, (((1,), (1,)), ((), ())),
                               preferred_element_type=F32)


def _encoder_kernel(nbatch, xT_ref, w1_ref, b1_ref, w2_ref, b2_ref, w3_ref,
                    b3_ref, wfc_ref, bfc_ref, w1f_ref, b1d_ref, d1bT_ref):
    h = _dot_tb(xT_ref[...], w1_ref[...]) + b1_ref[...]
    h = jnp.maximum(h, 0.0)
    h = jnp.maximum(_dot_tb(h, w2_ref[...]) + b2_ref[...], 0.0)
    h = _dot_tb(h, w3_ref[...]) + b3_ref[...]
    n = h.shape[0] // nbatch
    g = jnp.concatenate(
        [jnp.max(h[b * n:(b + 1) * n], axis=0, keepdims=True)
         for b in range(nbatch)], axis=0)
    feat = jnp.maximum(_dot_tb(g, wfc_ref[...]) + bfc_ref[...], 0.0)
    d1bT_ref[...] = _dot_tb(feat, w1f_ref[...]) + b1d_ref[...]


def _decoder_kernel(vert_ref, w1v_ref, d1b_ref, w2_ref, b2_ref, w3_ref,
                    b3_ref, w4_ref, b4_ref, out_ref, vb_ref):
    @pl.when(pl.program_id(1) == 0)
    def _():
        vb_ref[...] = jnp.dot(w1v_ref[0], vert_ref[...],
                              preferred_element_type=F32)

    h = jnp.maximum(vb_ref[...] + d1b_ref[0, 0], 0.0)
    h = jnp.maximum(
        jnp.dot(w2_ref[0], h, preferred_element_type=F32) + b2_ref[0], 0.0)
    h = jnp.maximum(
        jnp.dot(w3_ref[0], h, preferred_element_type=F32) + b3_ref[0], 0.0)
    o = jnp.dot(w4_ref[0], h, preferred_element_type=F32) + b4_ref[0]
    out_ref[0, 0] = 2.0 * jnp.tanh(o)


def kernel(x, enc_w1, enc_b1, enc_w2, enc_b2, enc_w3, enc_b3, fc_w, fc_b,
           dec_w1v, dec_w1f, dec_b1, dec_w2, dec_b2, dec_w3, dec_b3,
           dec_w4, dec_b4, vertex):
    B, _, N = x.shape
    P, C, _ = dec_w1v.shape
    V = vertex.shape[0]

    xT = jnp.transpose(x, (0, 2, 1)).reshape(B * N, 3)
    import functools
    d1bT = pl.pallas_call(
        functools.partial(_encoder_kernel, B),
        out_shape=jax.ShapeDtypeStruct((B, P * C), F32),
    )(xT, enc_w1, enc_b1.reshape(1, -1), enc_w2, enc_b2.reshape(1, -1),
      enc_w3, enc_b3.reshape(1, -1), fc_w, fc_b.reshape(1, -1),
      dec_w1f, dec_b1.reshape(1, -1))

    d1bc = jnp.transpose(d1bT.reshape(B, P, C), (1, 0, 2))[..., None]

    out4 = pl.pallas_call(
        _decoder_kernel,
        out_shape=jax.ShapeDtypeStruct((B, P, 3, V), F32),
        grid=(P, B),
        in_specs=[
            pl.BlockSpec((3, V), lambda p, b: (0, 0)),
            pl.BlockSpec((1, C, 3), lambda p, b: (p, 0, 0)),
            pl.BlockSpec((1, 1, C, 1), lambda p, b: (p, b, 0, 0)),
            pl.BlockSpec((1,) + dec_w2.shape[1:], lambda p, b: (p, 0, 0)),
            pl.BlockSpec((1,) + dec_b2.shape[1:], lambda p, b: (p, 0, 0)),
            pl.BlockSpec((1,) + dec_w3.shape[1:], lambda p, b: (p, 0, 0)),
            pl.BlockSpec((1,) + dec_b3.shape[1:], lambda p, b: (p, 0, 0)),
            pl.BlockSpec((1,) + dec_w4.shape[1:], lambda p, b: (p, 0, 0)),
            pl.BlockSpec((1,) + dec_b4.shape[1:], lambda p, b: (p, 0, 0)),
        ],
        out_specs=pl.BlockSpec((1, 1, 3, V), lambda p, b: (b, p, 0, 0)),
        scratch_shapes=[pltpu.VMEM((C, V), F32)],
        compiler_params=pltpu.CompilerParams(
            dimension_semantics=("parallel", "arbitrary")),
    )(jnp.transpose(vertex), dec_w1v, d1bc, dec_w2, dec_b2, dec_w3, dec_b3,
      dec_w4, dec_b4)

    return jnp.transpose(out4, (0, 1, 3, 2)).reshape(B, P * V, 3)
